# baseline (speedup 1.0000x reference)
import functools

import jax
import jax.numpy as jnp
from jax import lax
from jax.experimental import pallas as pl
from jax.experimental.pallas import tpu as pltpu
from jax.experimental.pallas import tpu_sc as plsc

B, S = 16384, 200
D = 32
N = B * S
NC, NS = 2, 16
NW = NC * NS
NPW = N // NW
C = 1024
NCHUNK = NPW // C
NBUF = 2
NSUPER = NCHUNK // NBUF

_mesh = plsc.VectorSubcoreMesh(core_axis_name="c", subcore_axis_name="s")


@functools.partial(
    pl.kernel,
    out_type=jax.ShapeDtypeStruct((N, D), jnp.float32),
    mesh=_mesh,
    scratch_types=[
        pltpu.VMEM((C,), jnp.int32),
        pltpu.VMEM((C,), jnp.int32),
        pltpu.VMEM((C, D), jnp.float32),
        pltpu.VMEM((C, D), jnp.float32),
        pltpu.SemaphoreType.DMA,
        pltpu.SemaphoreType.DMA,
        pltpu.SemaphoreType.DMA,
        pltpu.SemaphoreType.DMA,
        pltpu.SemaphoreType.DMA,
        pltpu.SemaphoreType.DMA,
    ],
    compiler_params=pltpu.CompilerParams(use_tc_tiling_on_sc=False),
)
def _gather(table_hbm, idx_hbm, out_hbm,
            idx0, idx1, rows0, rows1, is0, is1, gs0, gs1, os0, os1):
    idx_v = (idx0, idx1)
    rows_v = (rows0, rows1)
    isem = (is0, is1)
    gsem = (gs0, gs1)
    osem = (os0, os1)

    wid = lax.axis_index("s") * NC + lax.axis_index("c")
    base = wid * NPW

    def idx_src(ci):
        return idx_hbm.at[pl.ds(base + ci * C, C)]

    def out_dst(ci):
        return out_hbm.at[pl.ds(base + ci * C, C)]

    for b in range(NBUF):
        pltpu.async_copy(idx_src(b), idx_v[b], isem[b])

    def super_chunk(g, carry):
        for b in range(NBUF):
            ci = g * NBUF + b
            pltpu.make_async_copy(idx_src(ci), idx_v[b], isem[b]).wait()

            @pl.when(g > 0)
            def _():
                pltpu.make_async_copy(rows_v[b], out_dst(ci), osem[b]).wait()

            pltpu.async_copy(
                table_hbm.at[idx_v[b]], rows_v[b], gsem[b]
            ).wait()

            pltpu.async_copy(rows_v[b], out_dst(ci), osem[b])

            @pl.when(g < NSUPER - 1)
            def _():
                pltpu.async_copy(idx_src(ci + NBUF), idx_v[b], isem[b])
        return carry

    lax.fori_loop(0, NSUPER, super_chunk, 0)

    for b in range(NBUF):
        pltpu.make_async_copy(
            rows_v[b], out_dst(NCHUNK - NBUF + b), osem[b]
        ).wait()


def kernel(seqs, species, table):
    del species
    idx_flat = seqs.reshape(-1).astype(jnp.int32)
    out = _gather(table, idx_flat)
    return out.reshape(B, S, D)


# speedup vs baseline: 1.0014x; 1.0014x over previous
import functools

import jax
import jax.numpy as jnp
from jax import lax
from jax.experimental import pallas as pl
from jax.experimental.pallas import tpu as pltpu
from jax.experimental.pallas import tpu_sc as plsc

B, S = 16384, 200
D = 32
NC, NS = 2, 16
NW = NC * NS
BPW = B // NW
CS = 8
NCHUNK = BPW // CS
NBUF = 2
NSUPER = NCHUNK // NBUF

_mesh = plsc.VectorSubcoreMesh(core_axis_name="c", subcore_axis_name="s")


@functools.partial(
    pl.kernel,
    out_type=jax.ShapeDtypeStruct((B, S, D), jnp.float32),
    mesh=_mesh,
    scratch_types=[
        pltpu.VMEM((CS, S), jnp.int32),
        pltpu.VMEM((CS, S), jnp.int32),
        pltpu.VMEM((CS, S, D), jnp.float32),
        pltpu.VMEM((CS, S, D), jnp.float32),
        pltpu.SemaphoreType.DMA,
        pltpu.SemaphoreType.DMA,
        pltpu.SemaphoreType.DMA,
        pltpu.SemaphoreType.DMA,
        pltpu.SemaphoreType.DMA,
        pltpu.SemaphoreType.DMA,
    ],
    compiler_params=pltpu.CompilerParams(use_tc_tiling_on_sc=False),
)
def _gather(table_hbm, idx_hbm, out_hbm,
            idx0, idx1, rows0, rows1, is0, is1, gs0, gs1, os0, os1):
    idx_v = (idx0, idx1)
    rows_v = (rows0, rows1)
    isem = (is0, is1)
    gsem = (gs0, gs1)
    osem = (os0, os1)

    wid = lax.axis_index("s") * NC + lax.axis_index("c")
    base = wid * BPW

    def idx_src(ci):
        return idx_hbm.at[pl.ds(base + ci * CS, CS)]

    def out_dst(ci):
        return out_hbm.at[pl.ds(base + ci * CS, CS)]

    for b in range(NBUF):
        pltpu.async_copy(idx_src(b), idx_v[b], isem[b])

    def super_chunk(g, carry):
        for b in range(NBUF):
            ci = g * NBUF + b
            pltpu.make_async_copy(idx_src(ci), idx_v[b], isem[b]).wait()

            @pl.when(g > 0)
            def _():
                pltpu.make_async_copy(rows_v[b], out_dst(ci), osem[b]).wait()

            cps = [
                pltpu.async_copy(
                    table_hbm.at[idx_v[b].at[k]],
                    rows_v[b].at[k],
                    gsem[b],
                )
                for k in range(CS)
            ]
            for cp in cps:
                cp.wait()

            pltpu.async_copy(rows_v[b], out_dst(ci), osem[b])

            @pl.when(g < NSUPER - 1)
            def _():
                pltpu.async_copy(idx_src(ci + NBUF), idx_v[b], isem[b])
        return carry

    lax.fori_loop(0, NSUPER, super_chunk, 0)

    for b in range(NBUF):
        pltpu.make_async_copy(
            rows_v[b], out_dst(NCHUNK - NBUF + b), osem[b]
        ).wait()


def kernel(seqs, species, table):
    del species
    return _gather(table, seqs.astype(jnp.int32))
